# baseline (speedup 1.0000x reference)
import jax
import jax.numpy as jnp
from jax import lax
from jax.experimental import pallas as pl
from jax.experimental.pallas import tpu as pltpu
from jax.experimental.pallas import tpu_sc as plsc

Z_DIM = 128
EMBED_SIZE = 128
BATCH = 16384

NUM_CORES = 2
NUM_SUBCORES = 16
LANES = 16
NUM_WORKERS = NUM_CORES * NUM_SUBCORES
ROWS_PER_WORKER = BATCH // NUM_WORKERS
CHUNK = 128
CHUNKS_PER_WORKER = ROWS_PER_WORKER // CHUNK
NBUF = 3


def _vrsqrt(x):
    i = lax.bitcast_convert_type(x, jnp.int32)
    i = jnp.int32(0x5F3759DF) - lax.shift_right_arithmetic(i, 1)
    y = lax.bitcast_convert_type(i, jnp.float32)
    xhalf = x * 0.5
    for _ in range(2):
        y = y * (1.5 - xhalf * y * y)
    return y


def _normalize_chunk(rows_ref):
    @plsc.parallel_loop(0, CHUNK, 1, unroll=2)
    def _row(r):
        vs = []
        acc = None
        for k in range(EMBED_SIZE // LANES):
            v = rows_ref[r, pl.ds(k * LANES, LANES)]
            vs.append(v)
            acc = v * v if acc is None else acc + v * v
        for sh in (8, 4, 2, 1):
            perm = jnp.arange(LANES, dtype=jnp.int32) ^ sh
            acc = acc + acc.at[perm].get(mode="promise_in_bounds")
        rinv = _vrsqrt(acc)
        for k in range(EMBED_SIZE // LANES):
            rows_ref[r, pl.ds(k * LANES, LANES)] = vs[k] * rinv


def _body(z_hbm, y_hbm, emb_hbm, out_hbm, idx_v, rows_v, gsem, zsem, osem):
    wid = lax.axis_index("s") * NUM_CORES + lax.axis_index("c")
    row0 = wid * ROWS_PER_WORKER
    pltpu.sync_copy(
        y_hbm.at[pl.ds(wid * CHUNKS_PER_WORKER, CHUNKS_PER_WORKER)], idx_v)

    zcopies = []
    for j in range(CHUNKS_PER_WORKER):
        base = row0 + j * CHUNK
        zcopies.append(pltpu.async_copy(
            z_hbm.at[pl.ds(base, CHUNK)],
            out_hbm.at[pl.ds(base, CHUNK), pl.ds(0, Z_DIM)], zsem))

    gathers = [None] * CHUNKS_PER_WORKER
    stores = [None] * CHUNKS_PER_WORKER
    gathers[0] = pltpu.async_copy(emb_hbm.at[idx_v.at[0]], rows_v.at[0], gsem)
    for j in range(CHUNKS_PER_WORKER):
        buf = j % NBUF
        gathers[j].wait()
        if j + 1 < CHUNKS_PER_WORKER:
            if j + 1 >= NBUF:
                stores[j + 1 - NBUF].wait()
            gathers[j + 1] = pltpu.async_copy(
                emb_hbm.at[idx_v.at[j + 1]], rows_v.at[(j + 1) % NBUF], gsem)
        _normalize_chunk(rows_v.at[buf])
        stores[j] = pltpu.async_copy(
            rows_v.at[buf],
            out_hbm.at[pl.ds(row0 + j * CHUNK, CHUNK), pl.ds(Z_DIM, EMBED_SIZE)],
            osem)
    for j in range(max(0, CHUNKS_PER_WORKER - NBUF + 1), CHUNKS_PER_WORKER):
        stores[j].wait()
    for zc in zcopies:
        zc.wait()


@jax.jit
def kernel(z, y, embedding):
    y2 = y.reshape(BATCH // CHUNK, CHUNK)
    mesh = plsc.VectorSubcoreMesh(core_axis_name="c", subcore_axis_name="s",
                                  num_cores=NUM_CORES, num_subcores=NUM_SUBCORES)
    run = pl.kernel(
        _body,
        out_type=jax.ShapeDtypeStruct((BATCH, Z_DIM + EMBED_SIZE), jnp.float32),
        mesh=mesh,
        scratch_types=[
            pltpu.VMEM((CHUNKS_PER_WORKER, CHUNK), jnp.int32),
            pltpu.VMEM((NBUF, CHUNK, EMBED_SIZE), jnp.float32),
            pltpu.SemaphoreType.DMA,
            pltpu.SemaphoreType.DMA,
            pltpu.SemaphoreType.DMA,
        ],
    )
    return run(z, y2, embedding)


# speedup vs baseline: 7.7226x; 7.7226x over previous
import jax
import jax.numpy as jnp
from jax import lax
from jax.experimental import pallas as pl
from jax.experimental.pallas import tpu as pltpu
from jax.experimental.pallas import tpu_sc as plsc

Z_DIM = 128
EMBED_SIZE = 128
OUT_DIM = Z_DIM + EMBED_SIZE
BATCH = 16384

NUM_CORES = 2
NUM_SUBCORES = 16
LANES = 16
NUM_WORKERS = NUM_CORES * NUM_SUBCORES
ROWS_PER_WORKER = BATCH // NUM_WORKERS
CHUNK = 64
CHUNKS_PER_WORKER = ROWS_PER_WORKER // CHUNK
NBUF = 4


def _vrsqrt(x):
    i = lax.bitcast_convert_type(x, jnp.int32)
    i = jnp.int32(0x5F3759DF) - lax.shift_right_arithmetic(i, 1)
    y = lax.bitcast_convert_type(i, jnp.float32)
    xhalf = x * 0.5
    for _ in range(2):
        y = y * (1.5 - xhalf * y * y)
    return y


def _normalize_chunk(buf_ref):
    @plsc.parallel_loop(0, CHUNK, 1, unroll=2)
    def _row(r):
        vs = []
        acc = None
        for k in range(EMBED_SIZE // LANES):
            v = buf_ref[r, pl.ds(Z_DIM + k * LANES, LANES)]
            vs.append(v)
            acc = v * v if acc is None else acc + v * v
        for sh in (8, 4, 2, 1):
            perm = jnp.arange(LANES, dtype=jnp.int32) ^ sh
            acc = acc + acc.at[perm].get(mode="promise_in_bounds")
        rinv = _vrsqrt(acc)
        for k in range(EMBED_SIZE // LANES):
            buf_ref[r, pl.ds(Z_DIM + k * LANES, LANES)] = vs[k] * rinv


def _body(z_hbm, y_hbm, emb_hbm, out_hbm, idx_v, obuf_v, gsem, zsem, osem):
    wid = lax.axis_index("s") * NUM_CORES + lax.axis_index("c")
    row0 = wid * ROWS_PER_WORKER
    pltpu.sync_copy(
        y_hbm.at[pl.ds(wid * CHUNKS_PER_WORKER, CHUNKS_PER_WORKER)], idx_v)

    C, N = CHUNKS_PER_WORKER, NBUF
    gathers = [None] * C
    zin = [None] * C
    stores = [None] * C

    def start_inputs(k):
        base = row0 + k * CHUNK
        zin[k] = pltpu.async_copy(
            z_hbm.at[pl.ds(base, CHUNK)],
            obuf_v.at[k % N, :, pl.ds(0, Z_DIM)], zsem.at[k % N])
        gathers[k] = pltpu.async_copy(
            emb_hbm.at[idx_v.at[k]],
            obuf_v.at[k % N, :, pl.ds(Z_DIM, EMBED_SIZE)], gsem.at[k % N])

    for k in range(N):
        start_inputs(k)
    for j in range(C):
        if j >= 1 and j - 1 + N < C:
            stores[j - 1].wait()
            start_inputs(j - 1 + N)
        gathers[j].wait()
        zin[j].wait()
        _normalize_chunk(obuf_v.at[j % N])
        stores[j] = pltpu.async_copy(
            obuf_v.at[j % N], out_hbm.at[pl.ds(row0 + j * CHUNK, CHUNK)],
            osem.at[j % N])
    for j in range(C - N, C):
        stores[j].wait()


@jax.jit
def kernel(z, y, embedding):
    y2 = y.reshape(BATCH // CHUNK, CHUNK)
    mesh = plsc.VectorSubcoreMesh(core_axis_name="c", subcore_axis_name="s",
                                  num_cores=NUM_CORES, num_subcores=NUM_SUBCORES)
    run = pl.kernel(
        _body,
        out_type=jax.ShapeDtypeStruct((BATCH, OUT_DIM), jnp.float32),
        mesh=mesh,
        scratch_types=[
            pltpu.VMEM((CHUNKS_PER_WORKER, CHUNK), jnp.int32),
            pltpu.VMEM((NBUF, CHUNK, OUT_DIM), jnp.float32),
            pltpu.SemaphoreType.DMA((NBUF,)),
            pltpu.SemaphoreType.DMA((NBUF,)),
            pltpu.SemaphoreType.DMA((NBUF,)),
        ],
    )
    return run(z, y2, embedding)
